# baseline (speedup 1.0000x reference)
import functools

import jax
import jax.numpy as jnp
from jax import lax
from jax.experimental import pallas as pl
from jax.experimental.pallas import tpu as pltpu
from jax.experimental.pallas import tpu_sc as plsc

NODE_HIDDEN_ = 128
E_ = 4


def _interleave_sc(lp_e):
    B, E, npad = lp_e.shape
    n = npad - 1
    flat_len = E * n
    tab_w = E * npad
    num_cores, num_subcores = 2, 16
    nw = num_cores * num_subcores
    per_w = B // nw
    chunk = per_w * flat_len
    n_chunks = (chunk + 15) // 16
    mesh = plsc.VectorSubcoreMesh(core_axis_name="c", subcore_axis_name="s",
                                  num_cores=num_cores,
                                  num_subcores=num_subcores)

    @functools.partial(
        pl.kernel,
        out_type=jax.ShapeDtypeStruct((B * flat_len,), jnp.float32),
        mesh=mesh,
        scratch_types=[
            pltpu.VMEM((per_w * tab_w,), jnp.float32),
            pltpu.VMEM((16 * n_chunks,), jnp.float32),
        ],
        compiler_params=pltpu.CompilerParams(needs_layout_passes=False),
    )
    def interleave(lp_hbm, out_hbm, table_v, out_v):
        wid = lax.axis_index("s") * num_cores + lax.axis_index("c")
        for k in range(per_w):
            pltpu.sync_copy(lp_hbm.at[wid * per_w + k],
                            table_v.at[pl.ds(k * tab_w, tab_w)])
        lanes = lax.iota(jnp.int32, 16)

        def body(c, _):
            p = jnp.minimum(c * 16 + lanes, chunk - 1)
            in2 = p >= flat_len
            j = jnp.where(in2, p - flat_len, p)
            idx = (j & (E - 1)) * npad + (j >> 2)
            idx = jnp.where(in2, idx + tab_w, idx)
            out_v[pl.ds(c * 16, 16)] = plsc.load_gather(table_v, [idx])
            return 0

        lax.fori_loop(0, n_chunks, body, 0)
        pltpu.sync_copy(out_v.at[pl.ds(0, chunk)],
                        out_hbm.at[pl.ds(wid * chunk, chunk)])

    return interleave(lp_e.reshape(B, tab_w)).reshape(B, flat_len)


def _tc_body(d_enc_ref, hv_ref, W_ref, b_ref, lp_ref, chosen_ref):
    n_dests = hv_ref.shape[1] - 1
    hvb = hv_ref[0]
    dests = hvb[:n_dests, :]
    src = hvb[n_dests:, :]
    W = W_ref[...]
    Wd = W[:, :NODE_HIDDEN_]
    Ws = W[:, NODE_HIDDEN_:]
    sd = lax.dot_general(Wd, dests, (((1,), (1,)), ((), ())),
                         preferred_element_type=jnp.float32)
    ss = lax.dot_general(Ws, src, (((1,), (1,)), ((), ())),
                         preferred_element_type=jnp.float32)
    scores = sd + ss + b_ref[...]
    m = jnp.max(scores)
    ex = jnp.exp(scores - m)
    lse = m + jnp.log(jnp.sum(ex))
    lp = scores - lse
    lp_ref[0, :, pl.ds(0, n_dests)] = lp
    de = d_enc_ref[pl.program_id(0)]
    flat_idx = (lax.broadcasted_iota(jnp.int32, (E_, n_dests), 1) * E_
                + lax.broadcasted_iota(jnp.int32, (E_, n_dests), 0))
    chosen_ref[0, 0, 0] = jnp.sum(jnp.where(flat_idx == de, lp, 0.0))


def kernel(hv, d_enc, W, b):
    B, N, D = hv.shape
    n_dests = N - 1
    lp, chosen = pl.pallas_call(
        _tc_body,
        grid=(B,),
        in_specs=[
            pl.BlockSpec(memory_space=pltpu.SMEM),
            pl.BlockSpec((1, N, D), lambda i: (i, 0, 0)),
            pl.BlockSpec((E_, 2 * D), lambda i: (0, 0)),
            pl.BlockSpec((E_, 1), lambda i: (0, 0)),
        ],
        out_specs=[
            pl.BlockSpec((1, E_, N), lambda i: (i, 0, 0)),
            pl.BlockSpec((1, 1, 1), lambda i: (i, 0, 0),
                         memory_space=pltpu.SMEM),
        ],
        out_shape=[
            jax.ShapeDtypeStruct((B, E_, N), jnp.float32),
            jax.ShapeDtypeStruct((B, 1, 1), jnp.float32),
        ],
    )(d_enc, hv, W, b[:, None])
    lp_flat = _interleave_sc(lp)
    return lp_flat, chosen.reshape(B, 1)


# speedup vs baseline: 1.0055x; 1.0055x over previous
import functools

import jax
import jax.numpy as jnp
from jax import lax
from jax.experimental import pallas as pl
from jax.experimental.pallas import tpu as pltpu
from jax.experimental.pallas import tpu_sc as plsc

NODE_HIDDEN_ = 128
E_ = 4


def _interleave_sc(lp_e):
    B, E, npad = lp_e.shape
    n = npad - 1
    flat_len = E * n
    tab_w = E * npad
    num_cores, num_subcores = 2, 16
    nw = num_cores * num_subcores
    per_w = B // nw
    chunk = per_w * flat_len
    n_chunks = (chunk + 15) // 16
    mesh = plsc.VectorSubcoreMesh(core_axis_name="c", subcore_axis_name="s",
                                  num_cores=num_cores,
                                  num_subcores=num_subcores)

    u = 8
    cpg = tab_w // 16

    @functools.partial(
        pl.kernel,
        out_type=jax.ShapeDtypeStruct((B * flat_len,), jnp.float32),
        mesh=mesh,
        scratch_types=[
            pltpu.VMEM((per_w * tab_w + 16,), jnp.float32),
            pltpu.VMEM((per_w * tab_w,), jnp.float32),
            pltpu.SemaphoreType.DMA,
            pltpu.SemaphoreType.DMA,
        ],
        compiler_params=pltpu.CompilerParams(needs_layout_passes=False),
    )
    def interleave(lp_hbm, out_hbm, table_v, out_v, sem0, sem1):
        wid = lax.axis_index("s") * num_cores + lax.axis_index("c")
        cp0 = pltpu.async_copy(lp_hbm.at[wid * per_w],
                               table_v.at[pl.ds(0, tab_w)], sem0)
        cp1 = pltpu.async_copy(lp_hbm.at[wid * per_w + 1],
                               table_v.at[pl.ds(tab_w, tab_w)], sem1)
        lanes = lax.iota(jnp.int32, 16)
        e_base = (lanes & (E - 1)) * npad
        idx_a = e_base + (lanes >> 2)
        idx_b = e_base + ((lanes + 4) >> 2) + tab_w
        cp0.wait()

        def body_a(c, _):
            base = c * u
            for k in range(u):
                idx = idx_a + (base + k) * 4
                out_v[pl.ds((base + k) * 16, 16)] = plsc.load_gather(
                    table_v, [idx])
            return 0

        lax.fori_loop(0, cpg // u, body_a, 0)
        cp1.wait()

        def body_b(c, _):
            base = c * u
            for k in range(u):
                idx = idx_b + (base + k) * 4
                out_v[pl.ds(tab_w + (base + k) * 16, 16)] = plsc.load_gather(
                    table_v, [idx])
            return 0

        lax.fori_loop(0, cpg // u, body_b, 0)
        fix_idx = jnp.where(lanes < 12,
                            e_base + (n - 3) + (lanes >> 2),
                            (lanes - 12) * npad + tab_w)
        out_v[pl.ds(flat_len - 12, 16)] = plsc.load_gather(table_v, [fix_idx])
        pltpu.sync_copy(out_v.at[pl.ds(0, chunk)],
                        out_hbm.at[pl.ds(wid * chunk, chunk)])

    return interleave(lp_e.reshape(B, tab_w)).reshape(B, flat_len)


def _tc_body(d_enc_ref, hv_ref, W_ref, b_ref, lp_ref, chosen_ref):
    n_dests = hv_ref.shape[1] - 1
    hvb = hv_ref[0]
    dests = hvb[:n_dests, :]
    src = hvb[n_dests:, :]
    W = W_ref[...]
    Wd = W[:, :NODE_HIDDEN_]
    Ws = W[:, NODE_HIDDEN_:]
    sd = lax.dot_general(Wd, dests, (((1,), (1,)), ((), ())),
                         preferred_element_type=jnp.float32)
    ss = lax.dot_general(Ws, src, (((1,), (1,)), ((), ())),
                         preferred_element_type=jnp.float32)
    scores = sd + ss + b_ref[...]
    m = jnp.max(scores)
    ex = jnp.exp(scores - m)
    lse = m + jnp.log(jnp.sum(ex))
    lp = scores - lse
    lp_ref[0, :, pl.ds(0, n_dests)] = lp
    de = d_enc_ref[pl.program_id(0)]
    flat_idx = (lax.broadcasted_iota(jnp.int32, (E_, n_dests), 1) * E_
                + lax.broadcasted_iota(jnp.int32, (E_, n_dests), 0))
    chosen_ref[0, 0, 0] = jnp.sum(jnp.where(flat_idx == de, lp, 0.0))


def kernel(hv, d_enc, W, b):
    B, N, D = hv.shape
    n_dests = N - 1
    lp, chosen = pl.pallas_call(
        _tc_body,
        grid=(B,),
        in_specs=[
            pl.BlockSpec(memory_space=pltpu.SMEM),
            pl.BlockSpec((1, N, D), lambda i: (i, 0, 0)),
            pl.BlockSpec((E_, 2 * D), lambda i: (0, 0)),
            pl.BlockSpec((E_, 1), lambda i: (0, 0)),
        ],
        out_specs=[
            pl.BlockSpec((1, E_, N), lambda i: (i, 0, 0)),
            pl.BlockSpec((1, 1, 1), lambda i: (i, 0, 0),
                         memory_space=pltpu.SMEM),
        ],
        out_shape=[
            jax.ShapeDtypeStruct((B, E_, N), jnp.float32),
            jax.ShapeDtypeStruct((B, 1, 1), jnp.float32),
        ],
    )(d_enc, hv, W, b[:, None])
    lp_flat = _interleave_sc(lp)
    return lp_flat, chosen.reshape(B, 1)


# speedup vs baseline: 1.0156x; 1.0100x over previous
import functools

import jax
import jax.numpy as jnp
from jax import lax
from jax.experimental import pallas as pl
from jax.experimental.pallas import tpu as pltpu
from jax.experimental.pallas import tpu_sc as plsc

NODE_HIDDEN_ = 128
E_ = 4
_NUM_CORES, _NUM_SUBCORES = 2, 16


def _tc_body(hv_ref, W_ref, b_ref, lp_ref, lse_ref):
    n_dests = hv_ref.shape[1] - 1
    hvb = hv_ref[0]
    dests = hvb[:n_dests, :]
    src = hvb[n_dests:, :]
    W = W_ref[...]
    Wd = W[:, :NODE_HIDDEN_]
    Ws = W[:, NODE_HIDDEN_:]
    sd = lax.dot_general(Wd, dests, (((1,), (1,)), ((), ())),
                         preferred_element_type=jnp.float32)
    ss = lax.dot_general(Ws, src, (((1,), (1,)), ((), ())),
                         preferred_element_type=jnp.float32)
    scores = sd + ss + b_ref[...]
    m = jnp.max(scores)
    ex = jnp.exp(scores - m)
    lse = m + jnp.log(jnp.sum(ex))
    lp_ref[0] = scores - lse
    lse_ref[0, 0, 0] = lse


def _chosen_score_sc(hv, d_enc, W, b):
    B, N, D = hv.shape
    L = 16
    n_tiles = B // L
    mesh = plsc.VectorSubcoreMesh(core_axis_name="c", subcore_axis_name="s",
                                  num_cores=_NUM_CORES,
                                  num_subcores=_NUM_SUBCORES)
    d_enc_pad = jnp.pad(d_enc.reshape(n_tiles, L), ((0, 0), (0, 128 - L)))
    b_pad = jnp.pad(b, (0, L - E_))

    @functools.partial(
        pl.kernel,
        out_type=jax.ShapeDtypeStruct((n_tiles, 128), jnp.float32),
        mesh=mesh,
        scratch_types=[
            pltpu.VMEM((128,), jnp.int32),
            pltpu.VMEM((L,), jnp.int32),
            pltpu.VMEM((L,), jnp.int32),
            pltpu.VMEM((L,), jnp.int32),
            pltpu.VMEM((L, D), jnp.float32),
            pltpu.VMEM((L, D), jnp.float32),
            pltpu.VMEM((L, 2 * D), jnp.float32),
            pltpu.VMEM((L,), jnp.float32),
            pltpu.VMEM((128,), jnp.float32),
            pltpu.SemaphoreType.DMA,
            pltpu.SemaphoreType.DMA,
            pltpu.SemaphoreType.DMA,
        ],
        compiler_params=pltpu.CompilerParams(needs_layout_passes=False),
    )
    def chosen_score(hv_hbm, denc_hbm, w_hbm, b_hbm, out_hbm,
                     denc_v, didx_v, sidx_v, eidx_v, drows_v, srows_v,
                     wrows_v, b_v, out_v, sem0, sem1, sem2):
        wid = lax.axis_index("s") * _NUM_CORES + lax.axis_index("c")

        @pl.when(wid < n_tiles)
        def _():
            pltpu.sync_copy(denc_hbm.at[wid], denc_v)
            pltpu.sync_copy(b_hbm, b_v)
            lanes = lax.iota(jnp.int32, L)
            denc = denc_v[pl.ds(0, L)]
            base = (wid * L + lanes) * N
            didx_v[...] = base + (denc >> 2)
            sidx_v[...] = base + (N - 1)
            eidx_v[...] = denc & (E_ - 1)
            cp0 = pltpu.async_copy(hv_hbm.at[didx_v], drows_v, sem0)
            cp1 = pltpu.async_copy(hv_hbm.at[sidx_v], srows_v, sem1)
            cp2 = pltpu.async_copy(w_hbm.at[eidx_v], wrows_v, sem2)
            cp0.wait()
            cp1.wait()
            cp2.wait()

            def body(k, accs):
                acc = accs
                for u in range(4):
                    kk = k * 4 + u
                    col = jnp.full((L,), kk, jnp.int32)
                    dcol = plsc.load_gather(drows_v, [lanes, col])
                    scol = plsc.load_gather(srows_v, [lanes, col])
                    wd = plsc.load_gather(wrows_v, [lanes, col])
                    ws = plsc.load_gather(wrows_v, [lanes, col + D])
                    acc = acc + dcol * wd + scol * ws
                return acc

            acc = lax.fori_loop(0, D // 4, body, jnp.zeros((L,), jnp.float32))
            out_v[pl.ds(0, L)] = acc + plsc.load_gather(b_v, [eidx_v[...]])
            pltpu.sync_copy(out_v, out_hbm.at[wid])

    out = chosen_score(hv.reshape(B * N, D), d_enc_pad, W, b_pad)
    return out[:, :L].reshape(B, 1)


def kernel(hv, d_enc, W, b):
    B, N, D = hv.shape
    n_dests = N - 1
    sc_score = _chosen_score_sc(hv, d_enc, W, b)
    lp, lse = pl.pallas_call(
        _tc_body,
        grid=(B,),
        in_specs=[
            pl.BlockSpec((1, N, D), lambda i: (i, 0, 0)),
            pl.BlockSpec((E_, 2 * D), lambda i: (0, 0)),
            pl.BlockSpec((E_, 1), lambda i: (0, 0)),
        ],
        out_specs=[
            pl.BlockSpec((1, E_, n_dests), lambda i: (i, 0, 0)),
            pl.BlockSpec((1, 1, 1), lambda i: (i, 0, 0),
                         memory_space=pltpu.SMEM),
        ],
        out_shape=[
            jax.ShapeDtypeStruct((B, E_, n_dests), jnp.float32),
            jax.ShapeDtypeStruct((B, 1, 1), jnp.float32),
        ],
    )(hv, W, b[:, None])
    lp_flat = lp.transpose(0, 2, 1).reshape(B, n_dests * E_)
    chosen = sc_score - lse.reshape(B, 1)
    return lp_flat, chosen


# speedup vs baseline: 1.0157x; 1.0001x over previous
import functools

import jax
import jax.numpy as jnp
from jax import lax
from jax.experimental import pallas as pl
from jax.experimental.pallas import tpu as pltpu
from jax.experimental.pallas import tpu_sc as plsc

NODE_HIDDEN_ = 128
E_ = 4
_NUM_CORES, _NUM_SUBCORES = 2, 16


def _tc_body(hv_ref, W_ref, b_ref, lp_ref, lse_ref):
    n_dests = hv_ref.shape[1] - 1
    hvb = hv_ref[0]
    dests = hvb[:n_dests, :]
    src = hvb[n_dests:, :]
    W = W_ref[...]
    Wd = W[:, :NODE_HIDDEN_]
    Ws = W[:, NODE_HIDDEN_:]
    sd = lax.dot_general(Wd, dests, (((1,), (1,)), ((), ())),
                         preferred_element_type=jnp.float32)
    ss = lax.dot_general(Ws, src, (((1,), (1,)), ((), ())),
                         preferred_element_type=jnp.float32)
    scores = sd + ss + b_ref[...]
    m = jnp.max(scores)
    ex = jnp.exp(scores - m)
    lse = m + jnp.log(jnp.sum(ex))
    lp_ref[0] = scores - lse
    lse_ref[0, 0, 0] = lse


def _chosen_score_sc(hv, d_enc, W, b):
    B, N, D = hv.shape
    L = 16
    n_tiles = B // L
    mesh = plsc.VectorSubcoreMesh(core_axis_name="c", subcore_axis_name="s",
                                  num_cores=_NUM_CORES,
                                  num_subcores=_NUM_SUBCORES)
    d_enc_pad = jnp.pad(d_enc.reshape(n_tiles, L), ((0, 0), (0, 128 - L)))
    b_pad = jnp.pad(b, (0, L - E_))

    @functools.partial(
        pl.kernel,
        out_type=jax.ShapeDtypeStruct((n_tiles, 128), jnp.float32),
        mesh=mesh,
        scratch_types=[
            pltpu.VMEM((128,), jnp.int32),
            pltpu.VMEM((L,), jnp.int32),
            pltpu.VMEM((L,), jnp.int32),
            pltpu.VMEM((L,), jnp.int32),
            pltpu.VMEM((L, D), jnp.float32),
            pltpu.VMEM((L, D), jnp.float32),
            pltpu.VMEM((L, 2 * D), jnp.float32),
            pltpu.VMEM((L,), jnp.float32),
            pltpu.VMEM((128,), jnp.float32),
            pltpu.SemaphoreType.DMA,
            pltpu.SemaphoreType.DMA,
            pltpu.SemaphoreType.DMA,
        ],
        compiler_params=pltpu.CompilerParams(needs_layout_passes=False),
    )
    def chosen_score(hv_hbm, denc_hbm, w_hbm, b_hbm, out_hbm,
                     denc_v, didx_v, sidx_v, eidx_v, drows_v, srows_v,
                     wrows_v, b_v, out_v, sem0, sem1, sem2):
        wid = lax.axis_index("s") * _NUM_CORES + lax.axis_index("c")

        @pl.when(wid < n_tiles)
        def _():
            pltpu.sync_copy(denc_hbm.at[wid], denc_v)
            pltpu.sync_copy(b_hbm, b_v)
            lanes = lax.iota(jnp.int32, L)
            denc = denc_v[pl.ds(0, L)]
            base = (wid * L + lanes) * N
            didx_v[...] = base + (denc >> 2)
            sidx_v[...] = base + (N - 1)
            eidx_v[...] = denc & (E_ - 1)
            cp0 = pltpu.async_copy(hv_hbm.at[didx_v], drows_v, sem0)
            cp1 = pltpu.async_copy(hv_hbm.at[sidx_v], srows_v, sem1)
            cp2 = pltpu.async_copy(w_hbm.at[eidx_v], wrows_v, sem2)
            cp0.wait()
            cp1.wait()
            cp2.wait()

            def body(k, accs):
                acc = accs
                for u in range(4):
                    kk = k * 4 + u
                    col = jnp.full((L,), kk, jnp.int32)
                    dcol = plsc.load_gather(drows_v, [lanes, col])
                    scol = plsc.load_gather(srows_v, [lanes, col])
                    wd = plsc.load_gather(wrows_v, [lanes, col])
                    ws = plsc.load_gather(wrows_v, [lanes, col + D])
                    acc = acc + dcol * wd + scol * ws
                return acc

            acc = lax.fori_loop(0, D // 4, body, jnp.zeros((L,), jnp.float32))
            out_v[pl.ds(0, L)] = acc + plsc.load_gather(b_v, [eidx_v[...]])
            pltpu.sync_copy(out_v, out_hbm.at[wid])

    out = chosen_score(hv.reshape(B * N, D), d_enc_pad, W, b_pad)
    return out[:, :L].reshape(B, 1)


def kernel(hv, d_enc, W, b):
    B, N, D = hv.shape
    n_dests = N - 1
    sc_score = _chosen_score_sc(hv, d_enc, W, b)
    lp, lse = pl.pallas_call(
        _tc_body,
        grid=(B,),
        in_specs=[
            pl.BlockSpec((1, N, D), lambda i: (i, 0, 0)),
            pl.BlockSpec((E_, 2 * D), lambda i: (0, 0)),
            pl.BlockSpec((E_, 1), lambda i: (0, 0)),
        ],
        out_specs=[
            pl.BlockSpec((1, E_, n_dests), lambda i: (i, 0, 0)),
            pl.BlockSpec((1, 1, 1), lambda i: (i, 0, 0),
                         memory_space=pltpu.SMEM),
        ],
        out_shape=[
            jax.ShapeDtypeStruct((B, E_, n_dests), jnp.float32),
            jax.ShapeDtypeStruct((B, 1, 1), jnp.float32),
        ],
        compiler_params=pltpu.CompilerParams(skip_device_barrier=True),
    )(hv, W, b[:, None])
    lp_flat = lp.transpose(0, 2, 1).reshape(B, n_dests * E_)
    chosen = sc_score - lse.reshape(B, 1)
    return lp_flat, chosen


# speedup vs baseline: 1.0570x; 1.0407x over previous
import functools

import jax
import jax.numpy as jnp
from jax import lax
from jax.experimental import pallas as pl
from jax.experimental.pallas import tpu as pltpu
from jax.experimental.pallas import tpu_sc as plsc

NODE_HIDDEN_ = 128
E_ = 4
_NUM_CORES, _NUM_SUBCORES = 2, 16


def _tc_body(hv_ref, W_ref, b_ref, lp_ref, lse_ref):
    n_dests = hv_ref.shape[1] - 1
    hvb = hv_ref[0]
    dests = hvb[:n_dests, :]
    src = hvb[n_dests:, :]
    W = W_ref[...]
    Wd = W[:, :NODE_HIDDEN_]
    Ws = W[:, NODE_HIDDEN_:]
    sd = lax.dot_general(Wd, dests, (((1,), (1,)), ((), ())),
                         preferred_element_type=jnp.float32)
    ss = lax.dot_general(Ws, src, (((1,), (1,)), ((), ())),
                         preferred_element_type=jnp.float32)
    scores = sd + ss + b_ref[...]
    m = jnp.max(scores)
    ex = jnp.exp(scores - m)
    lse = m + jnp.log(jnp.sum(ex))
    lp_ref[0] = scores - lse
    lse_ref[0, 0, 0] = lse


def _chosen_score_sc(hv, d_enc, W, b):
    B, N, D = hv.shape
    L = 16
    n_tiles = B // L
    mesh = plsc.VectorSubcoreMesh(core_axis_name="c", subcore_axis_name="s",
                                  num_cores=_NUM_CORES,
                                  num_subcores=_NUM_SUBCORES)
    d_enc_pad = jnp.pad(d_enc.reshape(n_tiles, L), ((0, 0), (0, 128 - L)))
    b_pad = jnp.pad(b, (0, L - E_))

    @functools.partial(
        pl.kernel,
        out_type=jax.ShapeDtypeStruct((n_tiles, 128), jnp.float32),
        mesh=mesh,
        scratch_types=[
            pltpu.VMEM((128,), jnp.int32),
            pltpu.VMEM((L,), jnp.int32),
            pltpu.VMEM((L,), jnp.int32),
            pltpu.VMEM((L,), jnp.int32),
            pltpu.VMEM((L, D), jnp.float32),
            pltpu.VMEM((L, D), jnp.float32),
            pltpu.VMEM((L, 2 * D), jnp.float32),
            pltpu.VMEM((L,), jnp.float32),
            pltpu.VMEM((128,), jnp.float32),
            pltpu.SemaphoreType.DMA,
            pltpu.SemaphoreType.DMA,
            pltpu.SemaphoreType.DMA,
        ],
        compiler_params=pltpu.CompilerParams(needs_layout_passes=False),
    )
    def chosen_score(hv_hbm, denc_hbm, w_hbm, b_hbm, out_hbm,
                     denc_v, didx_v, sidx_v, eidx_v, drows_v, srows_v,
                     wrows_v, b_v, out_v, sem0, sem1, sem2):
        wid = lax.axis_index("s") * _NUM_CORES + lax.axis_index("c")

        @pl.when(wid < n_tiles)
        def _():
            pltpu.sync_copy(denc_hbm.at[wid], denc_v)
            pltpu.sync_copy(b_hbm, b_v)
            lanes = lax.iota(jnp.int32, L)
            denc = denc_v[pl.ds(0, L)]
            base = (wid * L + lanes) * N
            didx_v[...] = base + (denc >> 2)
            sidx_v[...] = base + (N - 1)
            eidx_v[...] = denc & (E_ - 1)
            cp0 = pltpu.async_copy(hv_hbm.at[didx_v], drows_v, sem0)
            cp1 = pltpu.async_copy(hv_hbm.at[sidx_v], srows_v, sem1)
            cp2 = pltpu.async_copy(w_hbm.at[eidx_v], wrows_v, sem2)
            cp0.wait()
            cp1.wait()
            cp2.wait()

            def body(k, accs):
                acc = accs
                for u in range(4):
                    kk = k * 4 + u
                    col = jnp.full((L,), kk, jnp.int32)
                    dcol = plsc.load_gather(drows_v, [lanes, col])
                    scol = plsc.load_gather(srows_v, [lanes, col])
                    wd = plsc.load_gather(wrows_v, [lanes, col])
                    ws = plsc.load_gather(wrows_v, [lanes, col + D])
                    acc = acc + dcol * wd + scol * ws
                return acc

            acc = lax.fori_loop(0, D // 4, body, jnp.zeros((L,), jnp.float32))
            out_v[pl.ds(0, L)] = acc + plsc.load_gather(b_v, [eidx_v[...]])
            pltpu.sync_copy(out_v, out_hbm.at[wid])

    out = chosen_score(hv.reshape(B * N, D), d_enc_pad, W, b_pad)
    return out[:, :L].reshape(B, 1)


def kernel(hv, d_enc, W, b):
    B, N, D = hv.shape
    n_dests = N - 1
    sc_score = _chosen_score_sc(hv, d_enc, W, b)
    lp, lse = pl.pallas_call(
        _tc_body,
        grid=(B,),
        in_specs=[
            pl.BlockSpec((1, N, D), lambda i: (i, 0, 0)),
            pl.BlockSpec((E_, 2 * D), lambda i: (0, 0)),
            pl.BlockSpec((E_, 1), lambda i: (0, 0)),
        ],
        out_specs=[
            pl.BlockSpec((1, E_, n_dests), lambda i: (i, 0, 0)),
            pl.BlockSpec((1, 1, 1), lambda i: (i, 0, 0),
                         memory_space=pltpu.SMEM),
        ],
        out_shape=[
            jax.ShapeDtypeStruct((B, E_, n_dests), jnp.float32),
            jax.ShapeDtypeStruct((B, 1, 1), jnp.float32),
        ],
        compiler_params=pltpu.CompilerParams(skip_device_barrier=True,
                                             vmem_limit_bytes=32 * 1024 * 1024),
    )(hv, W, b[:, None])
    lp_flat = lp.transpose(0, 2, 1).reshape(B, n_dests * E_)
    chosen = sc_score - lse.reshape(B, 1)
    return lp_flat, chosen
